# baseline (speedup 1.0000x reference)
import functools

import jax
import jax.numpy as jnp
from jax import lax
from jax.experimental import pallas as pl
from jax.experimental.pallas import tpu as pltpu
from jax.experimental.pallas import tpu_sc as plsc

NUM_CORES = 2
NUM_SUBCORES = 16
LANES = 16
NUM_WORKERS = NUM_CORES * NUM_SUBCORES

EMB_DIM = 64
BATCH = 16384
CHUNK = 128
B_PER_W = BATCH // NUM_WORKERS
CHUNKS_PER_W = B_PER_W // CHUNK
GROUPS_PER_W = B_PER_W // LANES


def _body(uids_hbm, iids_hbm, uemb_hbm, iemb_hbm, ubias_hbm, ibias_hbm,
          out_hbm, uid_v, iid_v, urows_v, irows_v, ub_v, ib_v, out_v,
          sem):
    wid = lax.axis_index("s") * NUM_CORES + lax.axis_index("c")
    base = wid * B_PER_W
    idx_row0 = wid * CHUNKS_PER_W

    pltpu.sync_copy(uids_hbm.at[pl.ds(idx_row0, CHUNKS_PER_W)], uid_v)
    pltpu.sync_copy(iids_hbm.at[pl.ds(idx_row0, CHUNKS_PER_W)], iid_v)

    copies = []
    for j in range(CHUNKS_PER_W):
        sl = pl.ds(j * CHUNK, CHUNK)
        copies.append(pltpu.async_copy(
            uemb_hbm.at[uid_v.at[j]], urows_v.at[sl], sem))
        copies.append(pltpu.async_copy(
            iemb_hbm.at[iid_v.at[j]], irows_v.at[sl], sem))
        copies.append(pltpu.async_copy(
            ubias_hbm.at[uid_v.at[j]], ub_v.at[sl], sem))
        copies.append(pltpu.async_copy(
            ibias_hbm.at[iid_v.at[j]], ib_v.at[sl], sem))
    for c in copies:
        c.wait()

    lane = lax.iota(jnp.int32, LANES)

    def group(g, carry):
        gsl = pl.ds(g * LANES, LANES)
        row = g * LANES + lane
        acc = ub_v[gsl] + ib_v[gsl]
        for d in range(EMB_DIM):
            col = jnp.full((LANES,), d, jnp.int32)
            uc = plsc.load_gather(urows_v, [row, col])
            vc = plsc.load_gather(irows_v, [row, col])
            acc = acc + uc * vc
        out_v[gsl] = acc
        return carry

    lax.fori_loop(0, GROUPS_PER_W, group, 0)

    pltpu.sync_copy(out_v, out_hbm.at[pl.ds(base, B_PER_W)])


@jax.jit
def _bpr_sc(uids, iids, user_emb, item_emb, ubias, ibias):
    mesh = plsc.VectorSubcoreMesh(
        core_axis_name="c", subcore_axis_name="s",
        num_cores=NUM_CORES, num_subcores=NUM_SUBCORES)
    return pl.kernel(
        _body,
        out_type=jax.ShapeDtypeStruct((BATCH,), jnp.float32),
        mesh=mesh,
        scratch_types=[
            pltpu.VMEM((CHUNKS_PER_W, CHUNK), jnp.int32),
            pltpu.VMEM((CHUNKS_PER_W, CHUNK), jnp.int32),
            pltpu.VMEM((B_PER_W, EMB_DIM), jnp.float32),
            pltpu.VMEM((B_PER_W, EMB_DIM), jnp.float32),
            pltpu.VMEM((B_PER_W,), jnp.float32),
            pltpu.VMEM((B_PER_W,), jnp.float32),
            pltpu.VMEM((B_PER_W,), jnp.float32),
            pltpu.SemaphoreType.DMA,
        ],
        compiler_params=pltpu.CompilerParams(
            needs_layout_passes=False, use_tc_tiling_on_sc=False),
    )(uids, iids, user_emb, item_emb, ubias, ibias)


def kernel(user_ids, item_ids, user_emb, item_emb, user_bias, item_bias):
    uids = user_ids.astype(jnp.int32).reshape(BATCH // CHUNK, CHUNK)
    iids = item_ids.astype(jnp.int32).reshape(BATCH // CHUNK, CHUNK)
    ubias = user_bias.reshape(-1)
    ibias = item_bias.reshape(-1)
    return _bpr_sc(uids, iids, user_emb, item_emb, ubias, ibias)
